# baseline (speedup 1.0000x reference)
import functools
import math

import jax
import jax.numpy as jnp
from jax import lax
from jax.experimental import pallas as pl
from jax.experimental.pallas import tpu as pltpu
from jax.experimental.pallas import tpu_sc as plsc

VOCAB = 1000000
D = 64
LANES = 16
NC = 2
NS = 16
NW = NC * NS

B_TOTAL = 4096 * 200
B_PER_W = B_TOTAL // NW
CHUNK = 512
NCHUNK = B_PER_W // CHUNK

SCALE = math.sqrt(D)


def _sc_body(table_hbm, idx_hbm, out_hbm, idx_v, rows_v, gsem):
    c = lax.axis_index("c")
    s = lax.axis_index("s")
    wid = s * NC + c
    base = wid * B_PER_W

    pltpu.sync_copy(idx_hbm.at[pl.ds(base, B_PER_W)], idx_v)

    @pl.loop(0, NCHUNK)
    def _chunk(i):
        off = i * CHUNK
        pltpu.async_copy(
            table_hbm.at[idx_v.at[pl.ds(off, CHUNK)]], rows_v, gsem
        ).wait()

        @pl.loop(0, CHUNK)
        def _row(r):
            for j in range(D // LANES):
                sl = pl.ds(j * LANES, LANES)
                rows_v[r, sl] = rows_v[r, sl] * SCALE

        pltpu.sync_copy(rows_v, out_hbm.at[pl.ds(base + off, CHUNK)])


@jax.jit
def _embed(idx_flat, table):
    mesh = plsc.VectorSubcoreMesh(
        core_axis_name="c", subcore_axis_name="s", num_cores=NC, num_subcores=NS
    )
    run = functools.partial(
        pl.kernel,
        out_type=jax.ShapeDtypeStruct((B_TOTAL, D), jnp.float32),
        mesh=mesh,
        scratch_types=[
            pltpu.VMEM((B_PER_W,), jnp.int32),
            pltpu.VMEM((CHUNK, D), jnp.float32),
            pltpu.SemaphoreType.DMA,
        ],
        compiler_params=pltpu.CompilerParams(use_tc_tiling_on_sc=False),
    )(_sc_body)
    return run(table, idx_flat)


def kernel(input_, table):
    idx_flat = input_.reshape(-1).astype(jnp.int32)
    out = _embed(idx_flat, table)
    return out.reshape(*input_.shape, D)
